# baseline (speedup 1.0000x reference)
import functools

import jax
import jax.numpy as jnp
from jax import lax
from jax.experimental import pallas as pl
from jax.experimental.pallas import tpu as pltpu
from jax.experimental.pallas import tpu_sc as plsc

_NEG = float("-inf")
_L = 16

_SORT8_NET = (
    (0, 1), (2, 3), (4, 5), (6, 7),
    (0, 2), (1, 3), (4, 6), (5, 7),
    (1, 2), (5, 6),
    (0, 4), (1, 5), (2, 6), (3, 7),
    (2, 4), (3, 5),
    (1, 2), (3, 4), (5, 6),
)

_CLEAN8_NET = (
    (0, 4), (1, 5), (2, 6), (3, 7),
    (0, 2), (1, 3), (4, 6), (5, 7),
    (0, 1), (2, 3), (4, 5), (6, 7),
)


def _cmpex(v, i, j):
    hi = jnp.maximum(v[i], v[j])
    lo = jnp.minimum(v[i], v[j])
    v[i] = hi
    v[j] = lo


def _sort8_desc(v):
    v = list(v)
    for i, j in _SORT8_NET:
        _cmpex(v, i, j)
    return v


def _merge_top8(r, s):
    m = [jnp.maximum(r[i], s[7 - i]) for i in range(8)]
    for i, j in _CLEAN8_NET:
        _cmpex(m, i, j)
    return m


_CHUNK = 256
_NCHUNK = 4096 // _CHUNK
_CB = 128


def _sc_topk(x):
    mesh = plsc.VectorSubcoreMesh(core_axis_name="c", subcore_axis_name="s")

    @functools.partial(
        pl.kernel,
        mesh=mesh,
        out_type=jax.ShapeDtypeStruct((4, 8, 1024), jnp.float32),
        scratch_types=[
            pltpu.VMEM((2, _CHUNK, _CB), jnp.float32),
            pltpu.VMEM((8, _CB), jnp.float32),
            pltpu.SemaphoreType.DMA,
            pltpu.SemaphoreType.DMA,
        ],
    )
    def sc_topk(x_hbm, out_hbm, buf, rst, sem0, sem1):
        cid = lax.axis_index("c")
        sid = lax.axis_index("s")
        wid = sid * 2 + cid
        b = wid // 8
        cbase = (wid % 8) * _CB
        sems = (sem0, sem1)

        neg = jnp.full((_L,), _NEG, jnp.float32)
        for k in range(8):
            for g in range(_CB // _L):
                rst[k, pl.ds(g * _L, _L)] = neg

        def start(chunk, slot):
            pltpu.make_async_copy(
                x_hbm.at[b, pl.ds(chunk * _CHUNK, _CHUNK), pl.ds(cbase, _CB)],
                buf.at[slot],
                sems[slot],
            ).start()

        def wait(slot):
            pltpu.make_async_copy(
                x_hbm.at[b, pl.ds(0, _CHUNK), pl.ds(cbase, _CB)],
                buf.at[slot],
                sems[slot],
            ).wait()

        def compute(slot):
            for g in range(_CB // _L):
                r = tuple(rst[k, pl.ds(g * _L, _L)] for k in range(8))

                def blk_body(i, r, _g=g, _slot=slot):
                    base = i * 8
                    v = [buf[_slot, base + j, pl.ds(_g * _L, _L)]
                         for j in range(8)]
                    v = _sort8_desc(v)
                    return tuple(_merge_top8(list(r), v))

                r = lax.fori_loop(0, _CHUNK // 8, blk_body, r)
                for k in range(8):
                    rst[k, pl.ds(g * _L, _L)] = r[k]

        start(0, 0)
        start(1, 1)

        def pair(i2, carry):
            wait(0)
            compute(0)
            start(2 * i2 + 2, 0)
            wait(1)
            compute(1)
            start(2 * i2 + 3, 1)
            return carry

        lax.fori_loop(0, _NCHUNK // 2 - 1, pair, 0)
        wait(0)
        compute(0)
        wait(1)
        compute(1)

        pltpu.sync_copy(rst, out_hbm.at[b, :, pl.ds(cbase, _CB)])

    return sc_topk(x)


def kernel(x):
    out = _sc_topk(x)
    return jnp.transpose(out, (0, 2, 1)).reshape(4, 8 * 1024)
